# Initial kernel scaffold; baseline (speedup 1.0000x reference)
import jax
import jax.numpy as jnp
from jax.experimental import pallas as pl


def kernel(x, W_rel, W_self, bias, edge_index, edge_type):
    raise NotImplementedError("write your pallas kernel here")


# speedup vs baseline: 3.2728x; 3.2728x over previous
import functools

import jax
import jax.numpy as jnp
from jax import lax
from jax.experimental import pallas as pl
from jax.experimental.pallas import tpu as pltpu
from jax.experimental.pallas import tpu_sc as plsc

_N = 10000
_D = 128
_E = 320000
_R = 4
_L = 3

_NC = 2
_NS = 16
_NW = _NC * _NS
_BLK = 128
_BPW = 80
_EPAD = _NW * _BPW * _BLK
_ACC = 10240
_RPS = _ACC // _NS
_TBL = (_R + 1) * _N

_MROWS = 1000
_MGRID = _N // _MROWS


def _mm_body(*refs, nparts, relu, matmul):
    parts = refs[:nparts]
    h = parts[0][...]
    for p in parts[1:nparts]:
        h = h + p[...]
    if relu:
        h = jnp.maximum(h, 0.0)
    if matmul:
        w_ref, b_ref, y_ref = refs[nparts:]
        y_ref[...] = jnp.dot(h, w_ref[...], preferred_element_type=jnp.float32) + b_ref[...]
    else:
        (y_ref,) = refs[nparts:]
        y_ref[...] = h


def _col0_spec():
    return pl.BlockSpec((_MROWS, _D), lambda i: (i, 0))


def _mm(parts, W=None, b=None, relu=False):
    nparts = len(parts)
    in_specs = [_col0_spec() for _ in parts]
    args = list(parts)
    if W is not None:
        in_specs += [pl.BlockSpec((_D, 5 * _D), lambda i: (0, 0)),
                     pl.BlockSpec((1, 5 * _D), lambda i: (0, 0))]
        args += [W, b.reshape(1, 5 * _D)]
        out_shape = jax.ShapeDtypeStruct((_N, 5 * _D), jnp.float32)
        out_spec = pl.BlockSpec((_MROWS, 5 * _D), lambda i: (i, 0))
    else:
        out_shape = jax.ShapeDtypeStruct((_N, _D), jnp.float32)
        out_spec = pl.BlockSpec((_MROWS, _D), lambda i: (i, 0))
    body = functools.partial(_mm_body, nparts=nparts, relu=relu, matmul=W is not None)
    return pl.pallas_call(body, grid=(_MGRID,), in_specs=in_specs,
                          out_specs=out_spec, out_shape=out_shape)(*args)


def _sc_body(table_hbm, gidx_hbm, dst_hbm, w16_hbm, out_hbm,
             gidx_v, dst_v, w16_v, rows_v, acc, sem):
    cid = lax.axis_index("c")
    sid = lax.axis_index("s")
    wid = cid * _NS + sid

    zero = jnp.zeros((16,), jnp.float32)

    def _zrow(i, carry):
        for c in range(_D // 16):
            rows_v[i, pl.ds(c * 16, 16)] = zero
        return carry

    lax.fori_loop(0, _BLK, _zrow, 0)
    nzb = _ACC // (_NS * _BLK)
    for j in range(nzb):
        pltpu.sync_copy(rows_v, acc.at[pl.ds((sid * nzb + j) * _BLK, _BLK)])
    plsc.subcore_barrier()

    def _blk(b, carry):
        pltpu.sync_copy(gidx_hbm.at[wid, b], gidx_v)
        pltpu.sync_copy(dst_hbm.at[wid, b], dst_v)
        pltpu.sync_copy(w16_hbm.at[wid, b], w16_v)
        pltpu.async_copy(table_hbm.at[gidx_v], rows_v, sem).wait()

        def _edge(e, c2):
            wv = w16_v[e, :]
            for c in range(_D // 16):
                sl = pl.ds(c * 16, 16)
                rows_v[e, sl] = rows_v[e, sl] * wv
            return c2

        lax.fori_loop(0, _BLK, _edge, 0)
        pltpu.sync_copy(rows_v, acc.at[dst_v], add=True)
        return carry

    lax.fori_loop(0, _BPW, _blk, 0)
    plsc.subcore_barrier()
    pltpu.sync_copy(acc.at[pl.ds(sid * _RPS, _RPS)],
                    out_hbm.at[cid, pl.ds(sid * _RPS, _RPS)])


def _sc_edge(table, gidx, dst, w16):
    mesh = plsc.VectorSubcoreMesh(core_axis_name="c", subcore_axis_name="s")
    kfn = pl.kernel(
        _sc_body,
        out_type=jax.ShapeDtypeStruct((_NC, _ACC, _D), jnp.float32),
        mesh=mesh,
        scratch_types=[
            pltpu.VMEM((_BLK,), jnp.int32),
            pltpu.VMEM((_BLK,), jnp.int32),
            pltpu.VMEM((_BLK, 16), jnp.float32),
            pltpu.VMEM((_BLK, _D), jnp.float32),
            pltpu.VMEM_SHARED((_ACC, _D), jnp.float32),
            pltpu.SemaphoreType.DMA,
        ],
    )
    return kfn(table, gidx, dst, w16)


def kernel(x, W_rel, W_self, bias, edge_index, edge_type):
    src = edge_index[0]
    dst = edge_index[1]
    et = edge_type

    key = et * _N + dst
    cnt = jax.ops.segment_sum(jnp.ones((_E,), jnp.float32), key,
                              num_segments=_R * _N)
    w = 1.0 / jnp.maximum(cnt[key], 1.0)

    gidx = src * (_R + 1) + 1 + et

    pad = _EPAD - _E
    gidx_p = jnp.pad(gidx, (0, pad)).reshape(_NW, _BPW, _BLK)
    dst_p = jnp.pad(dst, (0, pad)).reshape(_NW, _BPW, _BLK)
    w_p = jnp.pad(w, (0, pad))
    w16 = jnp.broadcast_to(w_p[:, None], (_EPAD, 16)).reshape(_NW, _BPW, _BLK, 16)

    W_all = jnp.concatenate([W_self[:, None], W_rel], axis=1)
    W_all = W_all.transpose(0, 2, 1, 3).reshape(_L, _D, 5 * _D)
    b_all = jnp.concatenate([bias, jnp.zeros((_L, _R * _D), jnp.float32)], axis=1)

    Y = _mm([x], W_all[0], b_all[0], relu=False)
    out = None
    for l in range(1, _L + 1):
        parts = _sc_edge(Y.reshape(_TBL, _D), gidx_p, dst_p, w16)
        p0, p1 = parts[0], parts[1]
        if l < _L:
            Y = _mm([Y, p0, p1], W_all[l], b_all[l], relu=True)
        else:
            out = _mm([Y, p0, p1], relu=False)
    return out


# speedup vs baseline: 3.9490x; 1.2066x over previous
import functools

import jax
import jax.numpy as jnp
from jax import lax
from jax.experimental import pallas as pl
from jax.experimental.pallas import tpu as pltpu
from jax.experimental.pallas import tpu_sc as plsc

_N = 10000
_D = 128
_E = 320000
_R = 4
_L = 3

_NC = 2
_NS = 16
_NW = _NC * _NS
_BLK = 128
_BPW = 80
_EPAD = _NW * _BPW * _BLK
_ACC = 10112
_RPS = _ACC // _NS
_TBL = (_R + 1) * _N

_MROWS = 1000
_MGRID = _N // _MROWS


def _mm_body(*refs, nparts, relu, matmul):
    parts = refs[:nparts]
    h = parts[0][...]
    for p in parts[1:nparts]:
        h = h + p[...]
    if relu:
        h = jnp.maximum(h, 0.0)
    if matmul:
        w_ref, b_ref, y_ref = refs[nparts:]
        y_ref[...] = jnp.dot(h, w_ref[...], preferred_element_type=jnp.float32) + b_ref[...]
    else:
        (y_ref,) = refs[nparts:]
        y_ref[...] = h


def _col0_spec():
    return pl.BlockSpec((_MROWS, _D), lambda i: (i, 0))


def _mm(parts, W=None, b=None, relu=False):
    nparts = len(parts)
    in_specs = [_col0_spec() for _ in parts]
    args = list(parts)
    if W is not None:
        in_specs += [pl.BlockSpec((_D, 5 * _D), lambda i: (0, 0)),
                     pl.BlockSpec((1, 5 * _D), lambda i: (0, 0))]
        args += [W, b.reshape(1, 5 * _D)]
        out_shape = jax.ShapeDtypeStruct((_N, 5 * _D), jnp.float32)
        out_spec = pl.BlockSpec((_MROWS, 5 * _D), lambda i: (i, 0))
    else:
        out_shape = jax.ShapeDtypeStruct((_N, _D), jnp.float32)
        out_spec = pl.BlockSpec((_MROWS, _D), lambda i: (i, 0))
    body = functools.partial(_mm_body, nparts=nparts, relu=relu, matmul=W is not None)
    return pl.pallas_call(body, grid=(_MGRID,), in_specs=in_specs,
                          out_specs=out_spec, out_shape=out_shape)(*args)


_GRP = 4
_NSG = _BPW // (2 * _GRP)


def _sc_body(table_hbm, pk_hbm, w_hbm, out_hbm,
             rows0, rows1, pk0, pk1, w0, w1, acc,
             gs0, gs1, ss0, ss1):
    cid = lax.axis_index("c")
    sid = lax.axis_index("s")
    wid = cid * _NS + sid

    rows = (rows0, rows1)
    gs = (gs0, gs1)
    ss = (ss0, ss1)
    meta = ((pk0, w0), (pk1, w1))

    zero = jnp.zeros((16,), jnp.float32)

    def _zrow(i, carry):
        for c in range(_D // 16):
            rows0[i, pl.ds(c * 16, 16)] = zero
        return carry

    lax.fori_loop(0, _BLK, _zrow, 0)
    nzb = -(-_ACC // (_NS * _BLK))
    for j in range(nzb):
        blkid = sid * nzb + j
        @pl.when(blkid * _BLK < _ACC)
        def _():
            pltpu.sync_copy(rows0, acc.at[pl.ds(blkid * _BLK, _BLK)])
    plsc.subcore_barrier()

    def _load_meta(m, g):
        pltpu.sync_copy(pk_hbm.at[wid, g], meta[m][0])
        pltpu.sync_copy(w_hbm.at[wid, g], meta[m][1])

    def _gather(m, k, p):
        return pltpu.make_async_copy(table_hbm.at[meta[m][0].at[k, 0]], rows[p], gs[p])

    def _scatter(m, k, p):
        return pltpu.make_async_copy(rows[p], acc.at[meta[m][0].at[k, 1]], ss[p])

    def _scale(m, k, p):
        rows_v = rows[p]
        w_v = meta[m][1]

        def _grp16(g, carry):
            wv16 = w_v[k, pl.ds(g * 16, 16)]
            for u in range(16):
                e = g * 16 + u
                wv = jnp.full((16,), wv16[u], dtype=jnp.float32)
                for c in range(_D // 16):
                    sl = pl.ds(c * 16, 16)
                    rows_v[e, sl] = rows_v[e, sl] * wv
            return carry

        lax.fori_loop(0, _BLK // 16, _grp16, 0)

    _load_meta(0, 0)
    _gather(0, 0, 0).start()

    def _sg(s, carry):
        for m in range(2):
            for k in range(_GRP):
                p = k % 2
                q = 1 - p
                if m == 0 and k == 0:
                    @pl.when(s >= 1)
                    def _():
                        _scatter(1, _GRP - 1, q).wait()
                    _load_meta(1, 2 * s + 1)
                elif k == 0:
                    _scatter(0, _GRP - 1, q).wait()
                    @pl.when(s < _NSG - 1)
                    def _():
                        _load_meta(0, 2 * s + 2)
                else:
                    _scatter(m, k - 1, q).wait()
                if k < _GRP - 1:
                    _gather(m, k + 1, q).start()
                elif m == 0:
                    _gather(1, 0, q).start()
                else:
                    @pl.when(s < _NSG - 1)
                    def _():
                        _gather(0, 0, q).start()
                _gather(m, k, p).wait()
                _scale(m, k, p)
                _scatter(m, k, p).start(add=True)
        return carry

    lax.fori_loop(0, _NSG, _sg, 0)
    _scatter(1, _GRP - 1, 1).wait()
    plsc.subcore_barrier()
    pltpu.sync_copy(acc.at[pl.ds(sid * _RPS, _RPS)],
                    out_hbm.at[cid, pl.ds(sid * _RPS, _RPS)])


def _sc_edge(table, pk, w):
    mesh = plsc.VectorSubcoreMesh(core_axis_name="c", subcore_axis_name="s")
    kfn = pl.kernel(
        _sc_body,
        out_type=jax.ShapeDtypeStruct((_NC, _ACC, _D), jnp.float32),
        mesh=mesh,
        scratch_types=[
            pltpu.VMEM((_BLK, _D), jnp.float32),
            pltpu.VMEM((_BLK, _D), jnp.float32),
            pltpu.VMEM((_GRP, 2, _BLK), jnp.int32),
            pltpu.VMEM((_GRP, 2, _BLK), jnp.int32),
            pltpu.VMEM((_GRP, _BLK), jnp.float32),
            pltpu.VMEM((_GRP, _BLK), jnp.float32),
            pltpu.VMEM_SHARED((_ACC, _D), jnp.float32),
            pltpu.SemaphoreType.DMA,
            pltpu.SemaphoreType.DMA,
            pltpu.SemaphoreType.DMA,
            pltpu.SemaphoreType.DMA,
        ],
    )
    return kfn(table, pk, w)


def kernel(x, W_rel, W_self, bias, edge_index, edge_type):
    src = edge_index[0]
    dst = edge_index[1]
    et = edge_type

    key = et * _N + dst
    cnt = jax.ops.segment_sum(jnp.ones((_E,), jnp.float32), key,
                              num_segments=_R * _N)
    w = 1.0 / jnp.maximum(cnt[key], 1.0)

    gidx = src * (_R + 1) + 1 + et

    pad = _EPAD - _E
    gidx_p = jnp.pad(gidx, (0, pad)).reshape(_NW, _BPW, _BLK)
    dst_p = jnp.pad(dst, (0, pad)).reshape(_NW, _BPW, _BLK)
    pk = jnp.stack([gidx_p, dst_p], axis=2)
    pk = pk.reshape(_NW, _BPW // _GRP, _GRP, 2, _BLK)
    wg = jnp.pad(w, (0, pad)).reshape(_NW, _BPW // _GRP, _GRP, _BLK)

    W_all = jnp.concatenate([W_self[:, None], W_rel], axis=1)
    W_all = W_all.transpose(0, 2, 1, 3).reshape(_L, _D, 5 * _D)
    b_all = jnp.concatenate([bias, jnp.zeros((_L, _R * _D), jnp.float32)], axis=1)

    Y = _mm([x], W_all[0], b_all[0], relu=False)
    out = None
    for l in range(1, _L + 1):
        parts = _sc_edge(Y.reshape(_TBL, _D), pk, wg)
        p0, p1 = parts[0], parts[1]
        if l < _L:
            Y = _mm([Y, p0, p1], W_all[l], b_all[l], relu=True)
        else:
            out = _mm([Y, p0, p1], relu=False)
    return out
